# Initial kernel scaffold; baseline (speedup 1.0000x reference)
import jax
import jax.numpy as jnp
from jax.experimental import pallas as pl


def kernel(attn_s):
    raise NotImplementedError("write your pallas kernel here")


# speedup vs baseline: 7.5714x; 7.5714x over previous
import functools

import jax
import jax.numpy as jnp
from jax import lax
from jax.experimental import pallas as pl
from jax.experimental.pallas import tpu as pltpu
from jax.experimental.pallas import tpu_sc as plsc

T = 2048
N = 4096
L = 16
NV = N // L
TOPK = 6
NC = 2
NS = 16
NW = NC * NS
ROWS = T // NW
EPS = 1e-7
NEG = -jnp.inf


def _sc_body(x_hbm, out_hbm, xv, wv):
    wid = lax.axis_index("s") * NC + lax.axis_index("c")
    base = wid * ROWS

    def do_row(r, _):
        row = base + r
        pltpu.sync_copy(x_hbm.at[row], xv)

        def p1(i, carry):
            t0, t1, t2, t3, t4, t5 = carry
            v = xv[pl.ds(i * L, L)]
            n0 = jnp.maximum(t0, v); v = jnp.minimum(t0, v)
            n1 = jnp.maximum(t1, v); v = jnp.minimum(t1, v)
            n2 = jnp.maximum(t2, v); v = jnp.minimum(t2, v)
            n3 = jnp.maximum(t3, v); v = jnp.minimum(t3, v)
            n4 = jnp.maximum(t4, v); v = jnp.minimum(t4, v)
            n5 = jnp.maximum(t5, v)
            return (n0, n1, n2, n3, n4, n5)

        neg = jnp.full((L,), NEG, jnp.float32)
        cands = lax.fori_loop(0, NV, p1, (neg,) * 6, unroll=8)

        def p2(j, carry):
            cnt, delta, c0, c1, c2, c3, c4, c5 = carry
            mv = jnp.maximum(jnp.maximum(jnp.maximum(c0, c1),
                                         jnp.maximum(c2, c3)),
                             jnp.maximum(c4, c5))
            m = jnp.max(mv)
            e0 = c0 == m; e1 = c1 == m; e2 = c2 == m
            e3 = c3 == m; e4 = c4 == m; e5 = c5 == m
            ones = jnp.ones((L,), jnp.int32)
            zeros = jnp.zeros((L,), jnp.int32)
            cv = (jnp.where(e0, ones, zeros) + jnp.where(e1, ones, zeros)
                  + jnp.where(e2, ones, zeros) + jnp.where(e3, ones, zeros)
                  + jnp.where(e4, ones, zeros) + jnp.where(e5, ones, zeros))
            c = jnp.sum(cv)
            delta = jnp.where((cnt < TOPK) & (cnt + c >= TOPK), m, delta)
            cnt = cnt + c
            c0 = jnp.where(e0, NEG, c0); c1 = jnp.where(e1, NEG, c1)
            c2 = jnp.where(e2, NEG, c2); c3 = jnp.where(e3, NEG, c3)
            c4 = jnp.where(e4, NEG, c4); c5 = jnp.where(e5, NEG, c5)
            return (cnt, delta, c0, c1, c2, c3, c4, c5)

        carry0 = (jnp.int32(0), jnp.float32(NEG)) + cands
        delta = lax.fori_loop(0, TOPK, p2, carry0)[1]

        def p3(i, acc):
            v = xv[pl.ds(i * L, L)]
            return acc + jnp.maximum(v - delta, 0.0)

        acc = lax.fori_loop(0, NV, p3, jnp.zeros((L,), jnp.float32),
                            unroll=8)
        s = jnp.sum(acc) + EPS
        inv = jnp.ones((L,), jnp.float32) / jnp.full((L,), s, jnp.float32)

        def p4(i, _):
            v = xv[pl.ds(i * L, L)]
            wv[pl.ds(i * L, L)] = jnp.maximum(v - delta, 0.0) * inv
            return 0

        lax.fori_loop(0, NV, p4, 0, unroll=8)
        pltpu.sync_copy(wv, out_hbm.at[row])
        return 0

    lax.fori_loop(0, ROWS, do_row, 0)


@jax.jit
def kernel(attn_s):
    x = attn_s.reshape(T, N)
    mesh = plsc.VectorSubcoreMesh(core_axis_name="c", subcore_axis_name="s")
    f = pl.kernel(
        _sc_body,
        out_type=jax.ShapeDtypeStruct((T, N), jnp.float32),
        mesh=mesh,
        scratch_types=[
            pltpu.VMEM((N,), jnp.float32),
            pltpu.VMEM((N,), jnp.float32),
        ],
        compiler_params=pltpu.CompilerParams(needs_layout_passes=False),
    )
    return f(x)


# speedup vs baseline: 11.4062x; 1.5065x over previous
import functools

import jax
import jax.numpy as jnp
from jax import lax
from jax.experimental import pallas as pl
from jax.experimental.pallas import tpu as pltpu
from jax.experimental.pallas import tpu_sc as plsc

T = 2048
N = 4096
L = 16
NV = N // L
TOPK = 6
NC = 2
NS = 16
NW = NC * NS
ROWS = T // NW
NPAIR = ROWS // 2
EPS = 1e-7
NEG = -jnp.inf


def _sc_body(x_hbm, out_hbm, xv0, xv1, wv0, wv1, si0, si1, so0, so1):
    wid = lax.axis_index("s") * NC + lax.axis_index("c")
    base = wid * ROWS

    pltpu.async_copy(x_hbm.at[base], xv0, si0)
    pltpu.async_copy(x_hbm.at[base + 1], xv1, si1)

    def compute_row(p, row, xv, wv, si, so):
        pltpu.make_async_copy(x_hbm.at[row], xv, si).wait()

        def p1(i, carry):
            t0, t1, t2, t3, t4, t5 = carry
            v = xv[pl.ds(i * L, L)]
            n0 = jnp.maximum(t0, v); v = jnp.minimum(t0, v)
            n1 = jnp.maximum(t1, v); v = jnp.minimum(t1, v)
            n2 = jnp.maximum(t2, v); v = jnp.minimum(t2, v)
            n3 = jnp.maximum(t3, v); v = jnp.minimum(t3, v)
            n4 = jnp.maximum(t4, v); v = jnp.minimum(t4, v)
            n5 = jnp.maximum(t5, v)
            return (n0, n1, n2, n3, n4, n5)

        neg = jnp.full((L,), NEG, jnp.float32)
        cands = lax.fori_loop(0, NV, p1, (neg,) * 6, unroll=8)

        def p2(j, carry):
            cnt, delta, c0, c1, c2, c3, c4, c5 = carry
            mv = jnp.maximum(jnp.maximum(jnp.maximum(c0, c1),
                                         jnp.maximum(c2, c3)),
                             jnp.maximum(c4, c5))
            m = jnp.max(mv)
            e0 = c0 == m; e1 = c1 == m; e2 = c2 == m
            e3 = c3 == m; e4 = c4 == m; e5 = c5 == m
            ones = jnp.ones((L,), jnp.int32)
            zeros = jnp.zeros((L,), jnp.int32)
            cv = (jnp.where(e0, ones, zeros) + jnp.where(e1, ones, zeros)
                  + jnp.where(e2, ones, zeros) + jnp.where(e3, ones, zeros)
                  + jnp.where(e4, ones, zeros) + jnp.where(e5, ones, zeros))
            c = jnp.sum(cv)
            delta = jnp.where((cnt < TOPK) & (cnt + c >= TOPK), m, delta)
            cnt = cnt + c
            c0 = jnp.where(e0, NEG, c0); c1 = jnp.where(e1, NEG, c1)
            c2 = jnp.where(e2, NEG, c2); c3 = jnp.where(e3, NEG, c3)
            c4 = jnp.where(e4, NEG, c4); c5 = jnp.where(e5, NEG, c5)
            return (cnt, delta, c0, c1, c2, c3, c4, c5)

        carry0 = (jnp.int32(0), jnp.float32(NEG)) + cands
        delta = lax.fori_loop(0, TOPK, p2, carry0)[1]

        sumv = (jnp.maximum(cands[0] - delta, 0.0)
                + jnp.maximum(cands[1] - delta, 0.0)
                + jnp.maximum(cands[2] - delta, 0.0)
                + jnp.maximum(cands[3] - delta, 0.0)
                + jnp.maximum(cands[4] - delta, 0.0)
                + jnp.maximum(cands[5] - delta, 0.0))
        s = jnp.sum(sumv) + EPS
        inv = jnp.ones((L,), jnp.float32) / jnp.full((L,), s, jnp.float32)

        @pl.when(p > 0)
        def _():
            pltpu.make_async_copy(wv, out_hbm.at[row - 2], so).wait()

        def p4(i, _):
            v = xv[pl.ds(i * L, L)]
            wv[pl.ds(i * L, L)] = jnp.maximum(v - delta, 0.0) * inv
            return 0

        lax.fori_loop(0, NV, p4, 0, unroll=8)
        pltpu.async_copy(wv, out_hbm.at[row], so)

        @pl.when(p < NPAIR - 1)
        def _():
            pltpu.async_copy(x_hbm.at[row + 2], xv, si)

    def pair_body(p, _):
        row0 = base + 2 * p
        compute_row(p, row0, xv0, wv0, si0, so0)
        compute_row(p, row0 + 1, xv1, wv1, si1, so1)
        return 0

    lax.fori_loop(0, NPAIR, pair_body, 0)
    pltpu.make_async_copy(wv0, out_hbm.at[base + ROWS - 2], so0).wait()
    pltpu.make_async_copy(wv1, out_hbm.at[base + ROWS - 1], so1).wait()


@jax.jit
def kernel(attn_s):
    x = attn_s.reshape(T, N)
    mesh = plsc.VectorSubcoreMesh(core_axis_name="c", subcore_axis_name="s")
    f = pl.kernel(
        _sc_body,
        out_type=jax.ShapeDtypeStruct((T, N), jnp.float32),
        mesh=mesh,
        scratch_types=[
            pltpu.VMEM((N,), jnp.float32),
            pltpu.VMEM((N,), jnp.float32),
            pltpu.VMEM((N,), jnp.float32),
            pltpu.VMEM((N,), jnp.float32),
            pltpu.SemaphoreType.DMA,
            pltpu.SemaphoreType.DMA,
            pltpu.SemaphoreType.DMA,
            pltpu.SemaphoreType.DMA,
        ],
        compiler_params=pltpu.CompilerParams(needs_layout_passes=False),
    )
    return f(x)


# speedup vs baseline: 24.2584x; 2.1268x over previous
import functools

import jax
import jax.numpy as jnp
from jax import lax
from jax.experimental import pallas as pl
from jax.experimental.pallas import tpu as pltpu
from jax.experimental.pallas import tpu_sc as plsc

T = 2048
N = 4096
L = 16
NV = N // L
TOPK = 6
NC = 2
NS = 16
NW = NC * NS
ROWS = T // NW
NPAIR = ROWS // 2
EPS = 1e-7
NEG = -jnp.inf


def _sc_body(x_hbm, out_hbm, xv0, xv1, wv0, wv1, si0, si1, so0, so1):
    wid = lax.axis_index("s") * NC + lax.axis_index("c")
    base = wid * ROWS

    pltpu.async_copy(x_hbm.at[base], xv0, si0)
    pltpu.async_copy(x_hbm.at[base + 1], xv1, si1)

    def compute_row(p, row, xv, wv, si, so):
        pltpu.make_async_copy(x_hbm.at[row], xv, si).wait()

        def p1(i, carry):
            t0, t1, t2, t3, t4, t5 = carry
            v = xv[pl.ds(i * L, L)]
            n0 = jnp.maximum(t0, v); v = jnp.minimum(t0, v)
            n1 = jnp.maximum(t1, v); v = jnp.minimum(t1, v)
            n2 = jnp.maximum(t2, v); v = jnp.minimum(t2, v)
            n3 = jnp.maximum(t3, v); v = jnp.minimum(t3, v)
            n4 = jnp.maximum(t4, v); v = jnp.minimum(t4, v)
            n5 = jnp.maximum(t5, v)
            return (n0, n1, n2, n3, n4, n5)

        neg = jnp.full((L,), NEG, jnp.float32)
        cands = lax.fori_loop(0, NV, p1, (neg,) * 6, unroll=8)

        def p2(j, carry):
            cnt, delta, c0, c1, c2, c3, c4, c5 = carry
            mv = jnp.maximum(jnp.maximum(jnp.maximum(c0, c1),
                                         jnp.maximum(c2, c3)),
                             jnp.maximum(c4, c5))
            m = jnp.max(mv)
            e0 = c0 == m; e1 = c1 == m; e2 = c2 == m
            e3 = c3 == m; e4 = c4 == m; e5 = c5 == m
            ones = jnp.ones((L,), jnp.int32)
            zeros = jnp.zeros((L,), jnp.int32)
            cv = (jnp.where(e0, ones, zeros) + jnp.where(e1, ones, zeros)
                  + jnp.where(e2, ones, zeros) + jnp.where(e3, ones, zeros)
                  + jnp.where(e4, ones, zeros) + jnp.where(e5, ones, zeros))
            c = jnp.sum(cv)
            delta = jnp.where((cnt < TOPK) & (cnt + c >= TOPK), m, delta)
            cnt = cnt + c
            c0 = jnp.where(e0, NEG, c0); c1 = jnp.where(e1, NEG, c1)
            c2 = jnp.where(e2, NEG, c2); c3 = jnp.where(e3, NEG, c3)
            c4 = jnp.where(e4, NEG, c4); c5 = jnp.where(e5, NEG, c5)
            return (cnt, delta, c0, c1, c2, c3, c4, c5)

        carry0 = (jnp.int32(0), jnp.float32(NEG)) + cands
        delta = lax.fori_loop(0, TOPK, p2, carry0)[1]

        sumv = (jnp.maximum(cands[0] - delta, 0.0)
                + jnp.maximum(cands[1] - delta, 0.0)
                + jnp.maximum(cands[2] - delta, 0.0)
                + jnp.maximum(cands[3] - delta, 0.0)
                + jnp.maximum(cands[4] - delta, 0.0)
                + jnp.maximum(cands[5] - delta, 0.0))
        s = jnp.sum(sumv) + EPS
        inv = jnp.ones((L,), jnp.float32) / jnp.full((L,), s, jnp.float32)

        @pl.when(p > 0)
        def _():
            pltpu.make_async_copy(wv, out_hbm.at[row - 2], so).wait()

        @plsc.parallel_loop(0, NV, step=1, unroll=8)
        def p4(i):
            v = xv[pl.ds(i * L, L)]
            wv[pl.ds(i * L, L)] = jnp.maximum(v - delta, 0.0) * inv
        pltpu.async_copy(wv, out_hbm.at[row], so)

        @pl.when(p < NPAIR - 1)
        def _():
            pltpu.async_copy(x_hbm.at[row + 2], xv, si)

    def pair_body(p, _):
        row0 = base + 2 * p
        compute_row(p, row0, xv0, wv0, si0, so0)
        compute_row(p, row0 + 1, xv1, wv1, si1, so1)
        return 0

    lax.fori_loop(0, NPAIR, pair_body, 0)
    pltpu.make_async_copy(wv0, out_hbm.at[base + ROWS - 2], so0).wait()
    pltpu.make_async_copy(wv1, out_hbm.at[base + ROWS - 1], so1).wait()


@jax.jit
def kernel(attn_s):
    x = attn_s.reshape(T, N)
    mesh = plsc.VectorSubcoreMesh(core_axis_name="c", subcore_axis_name="s")
    f = pl.kernel(
        _sc_body,
        out_type=jax.ShapeDtypeStruct((T, N), jnp.float32),
        mesh=mesh,
        scratch_types=[
            pltpu.VMEM((N,), jnp.float32),
            pltpu.VMEM((N,), jnp.float32),
            pltpu.VMEM((N,), jnp.float32),
            pltpu.VMEM((N,), jnp.float32),
            pltpu.SemaphoreType.DMA,
            pltpu.SemaphoreType.DMA,
            pltpu.SemaphoreType.DMA,
            pltpu.SemaphoreType.DMA,
        ],
        compiler_params=pltpu.CompilerParams(needs_layout_passes=False),
    )
    return f(x)


# speedup vs baseline: 26.2701x; 1.0829x over previous
import functools

import jax
import jax.numpy as jnp
from jax import lax
from jax.experimental import pallas as pl
from jax.experimental.pallas import tpu as pltpu
from jax.experimental.pallas import tpu_sc as plsc

T = 2048
N = 4096
L = 16
NV = N // L
TOPK = 6
NC = 2
NS = 16
NW = NC * NS
ROWS = T // NW
NPAIR = ROWS // 2
EPS = 1e-7
NEG = -jnp.inf


def _sc_body(x_hbm, out_hbm, xv0, xv1, wv0, wv1, si0, si1, so0, so1):
    wid = lax.axis_index("s") * NC + lax.axis_index("c")
    base = wid * ROWS

    pltpu.async_copy(x_hbm.at[base], xv0, si0)
    pltpu.async_copy(x_hbm.at[base + 1], xv1, si1)

    def knockout(cands):
        nvr = len(cands)

        def p2(j, carry):
            cnt, delta = carry[0], carry[1]
            cs = list(carry[2:])
            mv = cs[0]
            for c in cs[1:]:
                mv = jnp.maximum(mv, c)
            m = jnp.max(mv)
            es = [c == m for c in cs]
            ones = jnp.ones((L,), jnp.int32)
            zeros = jnp.zeros((L,), jnp.int32)
            cv = jnp.where(es[0], ones, zeros)
            for e in es[1:]:
                cv = cv + jnp.where(e, ones, zeros)
            c = jnp.sum(cv)
            delta = jnp.where((cnt < TOPK) & (cnt + c >= TOPK), m, delta)
            cnt = cnt + c
            cs = [jnp.where(e, NEG, c) for e, c in zip(es, cs)]
            return (cnt, delta, *cs)

        carry0 = (jnp.int32(0), jnp.float32(NEG)) + tuple(cands)
        delta = lax.fori_loop(0, TOPK, p2, carry0)[1]
        sumv = jnp.maximum(cands[0] - delta, 0.0)
        for c in cands[1:]:
            sumv = sumv + jnp.maximum(c - delta, 0.0)
        s = jnp.sum(sumv) + EPS
        return delta, s

    def compute_row(p, row, xv, wv, si, so):
        pltpu.make_async_copy(x_hbm.at[row], xv, si).wait()
        neg = jnp.full((L,), NEG, jnp.float32)

        def p1f(i, carry):
            t0, t1, t2 = carry
            v = xv[pl.ds(i * L, L)]
            n0 = jnp.maximum(t0, v); v = jnp.minimum(t0, v)
            n1 = jnp.maximum(t1, v); v = jnp.minimum(t1, v)
            n2 = jnp.maximum(t2, v)
            return (n0, n1, n2)

        c3 = lax.fori_loop(0, NV, p1f, (neg,) * 3, unroll=8)
        delta3, s3 = knockout(list(c3))

        safe = jnp.max(c3[2]) <= delta3

        def slow():
            def p1(i, carry):
                t0, t1, t2, t3, t4, t5 = carry
                v = xv[pl.ds(i * L, L)]
                n0 = jnp.maximum(t0, v); v = jnp.minimum(t0, v)
                n1 = jnp.maximum(t1, v); v = jnp.minimum(t1, v)
                n2 = jnp.maximum(t2, v); v = jnp.minimum(t2, v)
                n3 = jnp.maximum(t3, v); v = jnp.minimum(t3, v)
                n4 = jnp.maximum(t4, v); v = jnp.minimum(t4, v)
                n5 = jnp.maximum(t5, v)
                return (n0, n1, n2, n3, n4, n5)

            cands = lax.fori_loop(0, NV, p1, (neg,) * 6, unroll=8)
            return knockout(list(cands))

        delta, s = lax.cond(safe, lambda: (delta3, s3), slow)
        inv = jnp.ones((L,), jnp.float32) / jnp.full((L,), s, jnp.float32)

        @pl.when(p > 0)
        def _():
            pltpu.make_async_copy(wv, out_hbm.at[row - 2], so).wait()

        @plsc.parallel_loop(0, NV, step=1, unroll=8)
        def p4(i):
            v = xv[pl.ds(i * L, L)]
            wv[pl.ds(i * L, L)] = jnp.maximum(v - delta, 0.0) * inv
        pltpu.async_copy(wv, out_hbm.at[row], so)

        @pl.when(p < NPAIR - 1)
        def _():
            pltpu.async_copy(x_hbm.at[row + 2], xv, si)

    def pair_body(p, _):
        row0 = base + 2 * p
        compute_row(p, row0, xv0, wv0, si0, so0)
        compute_row(p, row0 + 1, xv1, wv1, si1, so1)
        return 0

    lax.fori_loop(0, NPAIR, pair_body, 0)
    pltpu.make_async_copy(wv0, out_hbm.at[base + ROWS - 2], so0).wait()
    pltpu.make_async_copy(wv1, out_hbm.at[base + ROWS - 1], so1).wait()


@jax.jit
def kernel(attn_s):
    x = attn_s.reshape(T, N)
    mesh = plsc.VectorSubcoreMesh(core_axis_name="c", subcore_axis_name="s")
    f = pl.kernel(
        _sc_body,
        out_type=jax.ShapeDtypeStruct((T, N), jnp.float32),
        mesh=mesh,
        scratch_types=[
            pltpu.VMEM((N,), jnp.float32),
            pltpu.VMEM((N,), jnp.float32),
            pltpu.VMEM((N,), jnp.float32),
            pltpu.VMEM((N,), jnp.float32),
            pltpu.SemaphoreType.DMA,
            pltpu.SemaphoreType.DMA,
            pltpu.SemaphoreType.DMA,
            pltpu.SemaphoreType.DMA,
        ],
        compiler_params=pltpu.CompilerParams(needs_layout_passes=False),
    )
    return f(x)


# speedup vs baseline: 27.3340x; 1.0405x over previous
import functools

import jax
import jax.numpy as jnp
from jax import lax
from jax.experimental import pallas as pl
from jax.experimental.pallas import tpu as pltpu
from jax.experimental.pallas import tpu_sc as plsc

T = 2048
N = 4096
L = 16
NV = N // L
TOPK = 6
NC = 2
NS = 16
NW = NC * NS
ROWS = T // NW
NPAIR = ROWS // 2
EPS = 1e-7
NEG = -jnp.inf


def _sc_body(x_hbm, out_hbm, xv0, xv1, wv0, wv1, si0, si1, so0, so1):
    wid = lax.axis_index("s") * NC + lax.axis_index("c")
    base = wid * ROWS

    pltpu.async_copy(x_hbm.at[base], xv0, si0)
    pltpu.async_copy(x_hbm.at[base + 1], xv1, si1)

    def _sortd(v):
        return plsc.sort_key_val(v, v, descending=True)[0]

    def _merge16(a, b):
        return _sortd(jnp.maximum(a, jnp.flip(b)))

    def topk_stats(sorted_cands):
        m = sorted_cands[0]
        for t in sorted_cands[1:]:
            m = _merge16(m, t)
        delta = m[TOPK - 1]
        sumv = jnp.maximum(m - delta, 0.0)
        s = jnp.sum(sumv) + EPS
        return delta, s

    def compute_row(p, row, xv, wv, si, so):
        pltpu.make_async_copy(x_hbm.at[row], xv, si).wait()
        neg = jnp.full((L,), NEG, jnp.float32)

        def p1f(i, carry):
            t0, t1, t2 = carry
            v = xv[pl.ds(i * L, L)]
            n0 = jnp.maximum(t0, v); v = jnp.minimum(t0, v)
            n1 = jnp.maximum(t1, v); v = jnp.minimum(t1, v)
            n2 = jnp.maximum(t2, v)
            return (n0, n1, n2)

        c3 = lax.fori_loop(0, NV, p1f, (neg,) * 3, unroll=8)
        sc3 = [_sortd(c) for c in c3]
        delta3, s3 = topk_stats(sc3)

        safe = sc3[2][0] <= delta3

        def slow():
            def p1(i, carry):
                t0, t1, t2, t3, t4, t5 = carry
                v = xv[pl.ds(i * L, L)]
                n0 = jnp.maximum(t0, v); v = jnp.minimum(t0, v)
                n1 = jnp.maximum(t1, v); v = jnp.minimum(t1, v)
                n2 = jnp.maximum(t2, v); v = jnp.minimum(t2, v)
                n3 = jnp.maximum(t3, v); v = jnp.minimum(t3, v)
                n4 = jnp.maximum(t4, v); v = jnp.minimum(t4, v)
                n5 = jnp.maximum(t5, v)
                return (n0, n1, n2, n3, n4, n5)

            cands = lax.fori_loop(0, NV, p1, (neg,) * 6, unroll=8)
            return topk_stats([_sortd(c) for c in cands])

        delta, s = lax.cond(safe, lambda: (delta3, s3), slow)
        inv = jnp.ones((L,), jnp.float32) / jnp.full((L,), s, jnp.float32)

        @pl.when(p > 0)
        def _():
            pltpu.make_async_copy(wv, out_hbm.at[row - 2], so).wait()

        @plsc.parallel_loop(0, NV, step=1, unroll=8)
        def p4(i):
            v = xv[pl.ds(i * L, L)]
            wv[pl.ds(i * L, L)] = jnp.maximum(v - delta, 0.0) * inv
        pltpu.async_copy(wv, out_hbm.at[row], so)

        @pl.when(p < NPAIR - 1)
        def _():
            pltpu.async_copy(x_hbm.at[row + 2], xv, si)

    def pair_body(p, _):
        row0 = base + 2 * p
        compute_row(p, row0, xv0, wv0, si0, so0)
        compute_row(p, row0 + 1, xv1, wv1, si1, so1)
        return 0

    lax.fori_loop(0, NPAIR, pair_body, 0)
    pltpu.make_async_copy(wv0, out_hbm.at[base + ROWS - 2], so0).wait()
    pltpu.make_async_copy(wv1, out_hbm.at[base + ROWS - 1], so1).wait()


@jax.jit
def kernel(attn_s):
    x = attn_s.reshape(T, N)
    mesh = plsc.VectorSubcoreMesh(core_axis_name="c", subcore_axis_name="s")
    f = pl.kernel(
        _sc_body,
        out_type=jax.ShapeDtypeStruct((T, N), jnp.float32),
        mesh=mesh,
        scratch_types=[
            pltpu.VMEM((N,), jnp.float32),
            pltpu.VMEM((N,), jnp.float32),
            pltpu.VMEM((N,), jnp.float32),
            pltpu.VMEM((N,), jnp.float32),
            pltpu.SemaphoreType.DMA,
            pltpu.SemaphoreType.DMA,
            pltpu.SemaphoreType.DMA,
            pltpu.SemaphoreType.DMA,
        ],
        compiler_params=pltpu.CompilerParams(needs_layout_passes=False),
    )
    return f(x)
